# baseline (speedup 1.0000x reference)
import jax
import jax.numpy as jnp
from jax.experimental import pallas as pl
from jax.experimental.pallas import tpu as pltpu

N = 100000
D_IN = 56
DP = 64
NUM_GRAPHS = 512
ROWS = 20000
STEPS = N // ROWS
HI = jax.lax.Precision.HIGHEST
DN0 = (((0,), (0,)), ((), ()))


def _fused_kernel(x_ref, batch_ref, a_ref, c_ref, W1a,
                  W2, b2, bn2_g, bn2_b, bn2_m, bn2_v,
                  W3, b3, bn3_g, bn3_b, bn3_m, bn3_v,
                  W4, b4, bn4_g, bn4_b, bn4_m, bn4_v,
                  out_ref, acc_ref):
    i = pl.program_id(0)

    @pl.when(i == 0)
    def _init():
        acc_ref[...] = jnp.zeros_like(acc_ref)

    x64 = jax.lax.pad(x_ref[...], 0.0, ((0, 0, 0), (0, DP - D_IN, 0)))
    h = x64 * a_ref[...] + c_ref[...]
    h = jnp.where(h > 0, h, jnp.exp(h) - 1.0)
    h_hi = h.astype(jnp.bfloat16)
    h_lo = (h - h_hi.astype(jnp.float32)).astype(jnp.bfloat16)

    seg = batch_ref[0]
    gid = jax.lax.broadcasted_iota(jnp.int16, (ROWS, NUM_GRAPHS), 1)
    onehot = jnp.where(seg == gid, jnp.bfloat16(1), jnp.bfloat16(0))
    acc_ref[...] += (
        jax.lax.dot_general(h_hi, onehot, DN0,
                            preferred_element_type=jnp.float32)
        + jax.lax.dot_general(h_lo, onehot, DN0,
                              preferred_element_type=jnp.float32))

    @pl.when(i == STEPS - 1)
    def _epilogue():
        pooled = jax.lax.dot_general(
            acc_ref[...], W1a[...], DN0,
            precision=HI, preferred_element_type=jnp.float32)
        z = jnp.dot(pooled, W2[...], precision=HI,
                    preferred_element_type=jnp.float32)
        z += b2[...]
        a2 = bn2_g[...] * jax.lax.rsqrt(bn2_v[...] + 1e-5)
        z = jnp.maximum(z * a2 + (bn2_b[...] - bn2_m[...] * a2), 0.0)
        z = jnp.dot(z, W3[...], precision=HI,
                    preferred_element_type=jnp.float32)
        z += b3[...]
        a3 = bn3_g[...] * jax.lax.rsqrt(bn3_v[...] + 1e-5)
        z = jnp.maximum(z * a3 + (bn3_b[...] - bn3_m[...] * a3), 0.0)
        z = jnp.dot(z, W4[...], precision=HI,
                    preferred_element_type=jnp.float32)
        z += b4[...]
        a4 = bn4_g[...] * jax.lax.rsqrt(bn4_v[...] + 1e-5)
        out_ref[...] = z * a4 + (bn4_b[...] - bn4_m[...] * a4)


def kernel(x, edge_index, batch,
           bn1_g, bn1_b, bn1_m, bn1_v, W1, b1,
           W2, b2, bn2_g, bn2_b, bn2_m, bn2_v,
           W3, b3, bn3_g, bn3_b, bn3_m, bn3_v,
           W4, b4, bn4_g, bn4_b, bn4_m, bn4_v):
    del edge_index
    batch3 = batch.astype(jnp.int16).reshape(STEPS, ROWS, 1)
    a = bn1_g * jax.lax.rsqrt(bn1_v + 1e-5)
    c = bn1_b - bn1_m * a
    a_pad = jnp.pad(a, (0, DP - D_IN)).reshape(1, DP)
    c_pad = jnp.pad(c, (0, DP - D_IN)).at[D_IN].set(1.0).reshape(1, DP)
    W1a = jnp.concatenate(
        [W1, b1[None, :], jnp.zeros((DP - D_IN - 1, 64), jnp.float32)], axis=0)
    row = lambda v: v.reshape(1, -1)

    full = lambda shape: pl.BlockSpec(shape, lambda i: (0,) * len(shape))
    out = pl.pallas_call(
        _fused_kernel,
        grid=(STEPS,),
        in_specs=[
            pl.BlockSpec((ROWS, D_IN), lambda i: (i, 0)),
            pl.BlockSpec((1, ROWS, 1), lambda i: (i, 0, 0)),
            full((1, DP)), full((1, DP)), full((DP, 64)),
            full(W2.shape), full((1, 128)),
            full((1, 128)), full((1, 128)), full((1, 128)), full((1, 128)),
            full(W3.shape), full((1, 64)),
            full((1, 64)), full((1, 64)), full((1, 64)), full((1, 64)),
            full(W4.shape), full((1, 1)),
            full((1, 1)), full((1, 1)), full((1, 1)), full((1, 1)),
        ],
        out_specs=pl.BlockSpec((NUM_GRAPHS, 1), lambda i: (0, 0)),
        out_shape=jax.ShapeDtypeStruct((NUM_GRAPHS, 1), jnp.float32),
        scratch_shapes=[
            pltpu.VMEM((DP, NUM_GRAPHS), jnp.float32),
        ],
    )(x, batch3, a_pad, c_pad, W1a,
      W2, row(b2), row(bn2_g), row(bn2_b), row(bn2_m), row(bn2_v),
      W3, row(b3), row(bn3_g), row(bn3_b), row(bn3_m), row(bn3_v),
      W4, row(b4), row(bn4_g), row(bn4_b), row(bn4_m), row(bn4_v))
    return out.reshape(-1)
